# Initial kernel scaffold; baseline (speedup 1.0000x reference)
import jax
import jax.numpy as jnp
from jax.experimental import pallas as pl


def kernel(x, W):
    raise NotImplementedError("write your pallas kernel here")


# speedup vs baseline: 2.6937x; 2.6937x over previous
import functools

import jax
import jax.numpy as jnp
from jax import lax
from jax.experimental import pallas as pl
from jax.experimental.pallas import tpu as pltpu

K_ROWS = 8192
D = 256
BATCH = 16384
COMMIT = 0.25


BJ = 2048
BK = 2048
NJ = K_ROWS // BJ
NK = K_ROWS // BK
BIG_I32 = 2 ** 30


def _kk(j, k):
    return lax.rem(j * (BJ // BK) + k, NK)


IDX_MASK = 0x7FF
SCORE_MASK = ~0x1FFF


QSCALE = 1270.0


def _argmin_body(wk_ref, wj_ref, idx_ref, wsq_ref, bv_ref, bi_ref):
    j = pl.program_id(0)
    k = pl.program_id(1)
    kk = _kk(j, k)
    wk = wk_ref[...]
    wqk = jnp.round(wk * QSCALE)
    wqj = jnp.round(wj_ref[...] * QSCALE)
    n2q = jnp.sum(wqk * wqk, axis=1,
                  keepdims=True).astype(jnp.int32)
    dots = lax.dot_general(
        wqk.astype(jnp.int8), wqj.astype(jnp.int8),
        dimension_numbers=(((1,), (1,)), ((), ())),
        preferred_element_type=jnp.int32)
    scores = (n2q + (1 << 24)) - 2 * dots
    bmin = jnp.min(scores, axis=0, keepdims=True)

    @pl.when(k == 0)
    def _():
        bv_ref[...] = jnp.full((1, BJ), 0x7FFFFFFF, jnp.int32)
        bi_ref[...] = jnp.zeros((1, BJ), jnp.int32)

    @pl.when(jnp.any(bmin < bv_ref[...]))
    def _():
        rowid = lax.broadcasted_iota(jnp.int32, (BK, BJ), 0)
        packed = (scores & SCORE_MASK) | rowid
        pmin = jnp.min(packed, axis=0, keepdims=True)
        gidx = (pmin & IDX_MASK) + kk * BK
        better = bmin < bv_ref[...]
        bv_ref[...] = jnp.where(better, bmin, bv_ref[...])
        bi_ref[...] = jnp.where(better, gidx, bi_ref[...])

    @pl.when(jnp.logical_and(j == 0, k == 0))
    def _():
        wsq_ref[0, 0] = 0.0

    @pl.when(j == 0)
    def _():
        n2k = jnp.sum(wk * wk, axis=1, keepdims=True)
        wsq_ref[0, 0] += jnp.sum(n2k)

    @pl.when(k == NK - 1)
    def _():
        idx_ref[...] = bi_ref[...]


def _assign_rows(W):
    return pl.pallas_call(
        _argmin_body,
        grid=(NJ, NK),
        in_specs=[
            pl.BlockSpec((BK, D), lambda j, k: (_kk(j, k), 0)),
            pl.BlockSpec((BJ, D), lambda j, k: (j, 0)),
        ],
        out_specs=[
            pl.BlockSpec((1, BJ), lambda j, k: (0, j)),
            pl.BlockSpec((1, 1), lambda j, k: (0, 0),
                         memory_space=pltpu.SMEM),
        ],
        out_shape=[
            jax.ShapeDtypeStruct((1, K_ROWS), jnp.int32),
            jax.ShapeDtypeStruct((1, 1), jnp.float32),
        ],
        scratch_shapes=[
            pltpu.VMEM((1, BJ), jnp.int32),
            pltpu.VMEM((1, BJ), jnp.int32),
        ],
    )(W, W)


from jax.experimental.pallas import tpu_sc as plsc

NW = 32
PW = BATCH // NW
SUB = 64
NSUB = PW // SUB
LANES = 16


def _gather_body(w_hbm, arow_hbm, x_hbm, quant_hbm, diff_hbm, part_hbm,
                 xs_v, ax_v, q_v, e_v, acc_v, sem, semw):
    c = lax.axis_index("c")
    s = lax.axis_index("s")
    wid = s * 2 + c
    base = wid * PW

    for t in range(NSUB):
        pltpu.sync_copy(x_hbm.at[pl.ds(base + t * SUB, SUB)], xs_v.at[t])

    copies = [pltpu.async_copy(arow_hbm.at[xs_v.at[t]], ax_v.at[t], sem)
              for t in range(NSUB)]
    for cp in copies:
        cp.wait()

    def fire(t):
        b = t % 2
        return (pltpu.async_copy(w_hbm.at[ax_v.at[t]], q_v.at[b], sem),
                pltpu.async_copy(w_hbm.at[xs_v.at[t]], e_v.at[b], sem))

    acc = jnp.zeros((LANES,), jnp.float32)
    pend = fire(0)
    writes = []
    for t in range(NSUB):
        b = t % 2
        cq, ce = pend
        cq.wait()
        ce.wait()
        writes.append(pltpu.async_copy(
            q_v.at[b], quant_hbm.at[pl.ds(base + t * SUB, SUB)], semw))
        if t + 1 < NSUB:
            if t >= 1:
                writes[2 * (t - 1)].wait()
                writes[2 * (t - 1) + 1].wait()
            pend = fire(t + 1)

        def row(r, a):
            for i in range(D // LANES):
                qv = q_v[b, r, pl.ds(i * LANES, LANES)]
                ev = e_v[b, r, pl.ds(i * LANES, LANES)]
                d = qv - ev
                e_v[b, r, pl.ds(i * LANES, LANES)] = d
                a = a + d * d
            return a

        acc = lax.fori_loop(0, SUB, row, acc)
        writes.append(pltpu.async_copy(
            e_v.at[b], diff_hbm.at[pl.ds(base + t * SUB, SUB)], semw))

    for cp in writes[2 * (NSUB - 2):]:
        cp.wait()
    acc_v[...] = acc
    pltpu.sync_copy(acc_v, part_hbm.at[wid])


def _gather_quantize(W, assign_row, x):
    mesh = plsc.VectorSubcoreMesh(core_axis_name="c", subcore_axis_name="s")
    f = functools.partial(
        pl.kernel,
        mesh=mesh,
        out_type=[
            jax.ShapeDtypeStruct((BATCH, D), jnp.float32),
            jax.ShapeDtypeStruct((BATCH, D), jnp.float32),
            jax.ShapeDtypeStruct((NW, LANES), jnp.float32),
        ],
        scratch_types=[
            pltpu.VMEM((NSUB, SUB), jnp.int32),
            pltpu.VMEM((NSUB, SUB), jnp.int32),
            pltpu.VMEM((2, SUB, D), jnp.float32),
            pltpu.VMEM((2, SUB, D), jnp.float32),
            pltpu.VMEM((LANES,), jnp.float32),
            pltpu.SemaphoreType.DMA,
            pltpu.SemaphoreType.DMA,
        ],
    )(_gather_body)
    return f(W, assign_row, x)


def kernel(x, W):
    x = x.astype(jnp.int32)
    idx2d, wsq = _assign_rows(W)
    assign_row = idx2d.reshape(K_ROWS)
    quantized, diff, part = _gather_quantize(W, assign_row, x)
    loss = jnp.sum(part) / jnp.float32(BATCH) + COMMIT * wsq[0, 0]
    return (loss, quantized, diff)


# speedup vs baseline: 3.3667x; 1.2498x over previous
import functools

import jax
import jax.numpy as jnp
from jax import lax
from jax.experimental import pallas as pl
from jax.experimental.pallas import tpu as pltpu

K_ROWS = 8192
D = 256
BATCH = 16384
COMMIT = 0.25


BJ = 2048
BK = 2048
NJ = K_ROWS // BJ
NK = K_ROWS // BK
BIG_I32 = 2 ** 30


def _kk(j, k):
    return lax.rem(j * (BJ // BK) + k, NK)


IDX_MASK = 0x7FF
SCORE_MASK = ~0x1FFF


def _argmin_body(wk_ref, wj_ref, idx_ref, wsq_ref, bv_ref, bi_ref):
    j = pl.program_id(0)
    k = pl.program_id(1)
    kk = _kk(j, k)
    wk = wk_ref[...]
    n2k = jnp.sum(wk * wk, axis=1, keepdims=True)
    dots = lax.dot_general(
        wk.astype(jnp.bfloat16), wj_ref[...].astype(jnp.bfloat16),
        dimension_numbers=(((1,), (1,)), ((), ())),
        preferred_element_type=jnp.float32)
    scores = (n2k + 4.0) - 2.0 * dots
    bmin = jnp.min(scores, axis=0, keepdims=True)

    @pl.when(k == 0)
    def _():
        bv_ref[...] = jnp.full((1, BJ), jnp.inf, jnp.float32)
        bi_ref[...] = jnp.zeros((1, BJ), jnp.int32)

    @pl.when(jnp.any(bmin < bv_ref[...]))
    def _():
        bits = lax.bitcast_convert_type(scores, jnp.int32)
        rowid = lax.broadcasted_iota(jnp.int32, (BK, BJ), 0)
        packed = (bits & SCORE_MASK) | rowid
        pmin = jnp.min(packed, axis=0, keepdims=True)
        gidx = (pmin & IDX_MASK) + kk * BK
        better = bmin < bv_ref[...]
        bv_ref[...] = jnp.where(better, bmin, bv_ref[...])
        bi_ref[...] = jnp.where(better, gidx, bi_ref[...])

    @pl.when(jnp.logical_and(j == 0, k == 0))
    def _():
        wsq_ref[0, 0] = 0.0

    @pl.when(j == 0)
    def _():
        wsq_ref[0, 0] += jnp.sum(n2k)

    @pl.when(k == NK - 1)
    def _():
        idx_ref[...] = bi_ref[...]


def _assign_rows(W):
    return pl.pallas_call(
        _argmin_body,
        grid=(NJ, NK),
        in_specs=[
            pl.BlockSpec((BK, D), lambda j, k: (_kk(j, k), 0)),
            pl.BlockSpec((BJ, D), lambda j, k: (j, 0)),
        ],
        out_specs=[
            pl.BlockSpec((1, BJ), lambda j, k: (0, j)),
            pl.BlockSpec((1, 1), lambda j, k: (0, 0),
                         memory_space=pltpu.SMEM),
        ],
        out_shape=[
            jax.ShapeDtypeStruct((1, K_ROWS), jnp.int32),
            jax.ShapeDtypeStruct((1, 1), jnp.float32),
        ],
        scratch_shapes=[
            pltpu.VMEM((1, BJ), jnp.float32),
            pltpu.VMEM((1, BJ), jnp.int32),
        ],
    )(W, W)


from jax.experimental.pallas import tpu_sc as plsc

NW = 32
PW = BATCH // NW
SUB = 64
NSUB = PW // SUB
LANES = 16


def _gather_body(w_hbm, arow_hbm, x_hbm, quant_hbm, diff_hbm, part_hbm,
                 xs_v, ax_v, q_v, e_v, acc_v, sem, semw):
    c = lax.axis_index("c")
    s = lax.axis_index("s")
    wid = s * 2 + c
    base = wid * PW

    pltpu.sync_copy(x_hbm.at[pl.ds(base, PW)], xs_v)

    copies = [pltpu.async_copy(arow_hbm.at[xs_v.at[pl.ds(i * 128, 128)]],
                               ax_v.at[pl.ds(i * 128, 128)], sem)
              for i in range(PW // 128)]
    for cp in copies:
        cp.wait()

    def fire(t):
        b = t % 2
        return (pltpu.async_copy(
                    w_hbm.at[ax_v.at[pl.ds(t * SUB, SUB)]], q_v.at[b], sem),
                pltpu.async_copy(
                    w_hbm.at[xs_v.at[pl.ds(t * SUB, SUB)]], e_v.at[b], sem))

    acc = jnp.zeros((LANES,), jnp.float32)
    pend = fire(0)
    writes = []
    for t in range(NSUB):
        b = t % 2
        cq, ce = pend
        cq.wait()
        ce.wait()
        writes.append(pltpu.async_copy(
            q_v.at[b], quant_hbm.at[pl.ds(base + t * SUB, SUB)], semw))
        if t + 1 < NSUB:
            if t >= 1:
                writes[2 * (t - 1)].wait()
                writes[2 * (t - 1) + 1].wait()
            pend = fire(t + 1)

        def row(r, a):
            for i in range(D // LANES):
                qv = q_v[b, r, pl.ds(i * LANES, LANES)]
                ev = e_v[b, r, pl.ds(i * LANES, LANES)]
                d = qv - ev
                e_v[b, r, pl.ds(i * LANES, LANES)] = d
                a = a + d * d
            return a

        acc = lax.fori_loop(0, SUB, row, acc)
        writes.append(pltpu.async_copy(
            e_v.at[b], diff_hbm.at[pl.ds(base + t * SUB, SUB)], semw))

    for cp in writes[2 * (NSUB - 2):]:
        cp.wait()
    acc_v[...] = acc
    pltpu.sync_copy(acc_v, part_hbm.at[wid])


def _gather_quantize(W, assign_row, x):
    mesh = plsc.VectorSubcoreMesh(core_axis_name="c", subcore_axis_name="s")
    f = functools.partial(
        pl.kernel,
        mesh=mesh,
        out_type=[
            jax.ShapeDtypeStruct((BATCH, D), jnp.float32),
            jax.ShapeDtypeStruct((BATCH, D), jnp.float32),
            jax.ShapeDtypeStruct((NW, LANES), jnp.float32),
        ],
        scratch_types=[
            pltpu.VMEM((PW,), jnp.int32),
            pltpu.VMEM((PW,), jnp.int32),
            pltpu.VMEM((2, SUB, D), jnp.float32),
            pltpu.VMEM((2, SUB, D), jnp.float32),
            pltpu.VMEM((LANES,), jnp.float32),
            pltpu.SemaphoreType.DMA,
            pltpu.SemaphoreType.DMA,
        ],
    )(_gather_body)
    return f(W, assign_row, x)


def kernel(x, W):
    x = x.astype(jnp.int32)
    idx2d, wsq = _assign_rows(W)
    assign_row = idx2d.reshape(K_ROWS)
    quantized, diff, part = _gather_quantize(W, assign_row, x)
    loss = jnp.sum(part) / jnp.float32(BATCH) + COMMIT * wsq[0, 0]
    return (loss, quantized, diff)


# speedup vs baseline: 3.3733x; 1.0020x over previous
import functools

import jax
import jax.numpy as jnp
from jax import lax
from jax.experimental import pallas as pl
from jax.experimental.pallas import tpu as pltpu

K_ROWS = 8192
D = 256
BATCH = 16384
COMMIT = 0.25


BJ = 2048
BK = 2048
NJ = K_ROWS // BJ
NK = K_ROWS // BK
BIG_I32 = 2 ** 30


def _kk(j, k):
    return lax.rem(j * (BJ // BK) + k, NK)


IDX_MASK = 0x7FF
SCORE_MASK = ~0x1FFF


def _argmin_body(wk_ref, wj_ref, idx_ref, wsq_ref, bv_ref, bi_ref):
    j = pl.program_id(0)
    k = pl.program_id(1)
    kk = _kk(j, k)
    wk = wk_ref[...]
    n2k = jnp.sum(wk * wk, axis=1, keepdims=True)
    dots = lax.dot_general(
        wk.astype(jnp.bfloat16), wj_ref[...].astype(jnp.bfloat16),
        dimension_numbers=(((1,), (1,)), ((), ())),
        preferred_element_type=jnp.float32)
    scores = (n2k + 4.0) - 2.0 * dots
    bmin = jnp.min(scores, axis=0, keepdims=True)

    @pl.when(k == 0)
    def _():
        bv_ref[...] = jnp.full((1, BJ), jnp.inf, jnp.float32)
        bi_ref[...] = jnp.zeros((1, BJ), jnp.int32)

    @pl.when(jnp.any(bmin < bv_ref[...]))
    def _():
        bits = lax.bitcast_convert_type(scores, jnp.int32)
        rowid = lax.broadcasted_iota(jnp.int32, (BK, BJ), 0)
        packed = (bits & SCORE_MASK) | rowid
        pmin = jnp.min(packed, axis=0, keepdims=True)
        gidx = (pmin & IDX_MASK) + kk * BK
        better = bmin < bv_ref[...]
        bv_ref[...] = jnp.where(better, bmin, bv_ref[...])
        bi_ref[...] = jnp.where(better, gidx, bi_ref[...])

    @pl.when(jnp.logical_and(j == 0, k == 0))
    def _():
        wsq_ref[0, 0] = 0.0

    @pl.when(j == 0)
    def _():
        wsq_ref[0, 0] += jnp.sum(n2k)

    @pl.when(k == NK - 1)
    def _():
        idx_ref[...] = bi_ref[...]


def _assign_rows(W):
    return pl.pallas_call(
        _argmin_body,
        grid=(NJ, NK),
        in_specs=[
            pl.BlockSpec((BK, D), lambda j, k: (_kk(j, k), 0)),
            pl.BlockSpec((BJ, D), lambda j, k: (j, 0)),
        ],
        out_specs=[
            pl.BlockSpec((1, BJ), lambda j, k: (0, j)),
            pl.BlockSpec((1, 1), lambda j, k: (0, 0),
                         memory_space=pltpu.SMEM),
        ],
        out_shape=[
            jax.ShapeDtypeStruct((1, K_ROWS), jnp.int32),
            jax.ShapeDtypeStruct((1, 1), jnp.float32),
        ],
        scratch_shapes=[
            pltpu.VMEM((1, BJ), jnp.float32),
            pltpu.VMEM((1, BJ), jnp.int32),
        ],
    )(W, W)


from jax.experimental.pallas import tpu_sc as plsc

NW = 32
PW = BATCH // NW
SUB = 64
NSUB = PW // SUB
LANES = 16


def _gather_body(w_hbm, arow_hbm, x_hbm, quant_hbm, diff_hbm, part_hbm,
                 xs_v, ax_v, q_v, e_v, acc_v, sem, semw):
    c = lax.axis_index("c")
    s = lax.axis_index("s")
    wid = s * 2 + c
    base = wid * PW

    pltpu.sync_copy(x_hbm.at[pl.ds(base, PW)], xs_v)

    copies = [pltpu.async_copy(arow_hbm.at[xs_v.at[pl.ds(i * 128, 128)]],
                               ax_v.at[pl.ds(i * 128, 128)], sem)
              for i in range(PW // 128)]
    for cp in copies:
        cp.wait()

    def fire(t):
        b = t % 2
        return (pltpu.async_copy(
                    w_hbm.at[ax_v.at[pl.ds(t * SUB, SUB)]], q_v.at[b], sem),
                pltpu.async_copy(
                    w_hbm.at[xs_v.at[pl.ds(t * SUB, SUB)]], e_v.at[b], sem))

    acc = tuple(jnp.zeros((LANES,), jnp.float32)
                for _ in range(D // LANES))
    pend = fire(0)
    writes = []
    for t in range(NSUB):
        b = t % 2
        cq, ce = pend
        cq.wait()
        ce.wait()
        writes.append(pltpu.async_copy(
            q_v.at[b], quant_hbm.at[pl.ds(base + t * SUB, SUB)], semw))
        if t + 1 < NSUB:
            if t >= 1:
                writes[2 * (t - 1)].wait()
                writes[2 * (t - 1) + 1].wait()
            pend = fire(t + 1)

        def row(r, accs):
            out = []
            for i in range(D // LANES):
                qv = q_v[b, r, pl.ds(i * LANES, LANES)]
                ev = e_v[b, r, pl.ds(i * LANES, LANES)]
                d = qv - ev
                e_v[b, r, pl.ds(i * LANES, LANES)] = d
                out.append(accs[i] + d * d)
            return tuple(out)

        acc = lax.fori_loop(0, SUB, row, acc)
        writes.append(pltpu.async_copy(
            e_v.at[b], diff_hbm.at[pl.ds(base + t * SUB, SUB)], semw))

    for cp in writes[2 * (NSUB - 2):]:
        cp.wait()
    total = acc[0]
    for a in acc[1:]:
        total = total + a
    acc_v[...] = total
    pltpu.sync_copy(acc_v, part_hbm.at[wid])


def _gather_quantize(W, assign_row, x):
    mesh = plsc.VectorSubcoreMesh(core_axis_name="c", subcore_axis_name="s")
    f = functools.partial(
        pl.kernel,
        mesh=mesh,
        out_type=[
            jax.ShapeDtypeStruct((BATCH, D), jnp.float32),
            jax.ShapeDtypeStruct((BATCH, D), jnp.float32),
            jax.ShapeDtypeStruct((NW, LANES), jnp.float32),
        ],
        scratch_types=[
            pltpu.VMEM((PW,), jnp.int32),
            pltpu.VMEM((PW,), jnp.int32),
            pltpu.VMEM((2, SUB, D), jnp.float32),
            pltpu.VMEM((2, SUB, D), jnp.float32),
            pltpu.VMEM((LANES,), jnp.float32),
            pltpu.SemaphoreType.DMA,
            pltpu.SemaphoreType.DMA,
        ],
    )(_gather_body)
    return f(W, assign_row, x)


def kernel(x, W):
    x = x.astype(jnp.int32)
    idx2d, wsq = _assign_rows(W)
    assign_row = idx2d.reshape(K_ROWS)
    quantized, diff, part = _gather_quantize(W, assign_row, x)
    loss = jnp.sum(part) / jnp.float32(BATCH) + COMMIT * wsq[0, 0]
    return (loss, quantized, diff)
